# baseline (speedup 1.0000x reference)
import functools

import jax
import jax.numpy as jnp
from jax import lax
from jax.experimental import pallas as pl
from jax.experimental.pallas import tpu as pltpu
from jax.experimental.pallas import tpu_sc as plsc

N = 10000
E = 320000
D = 128

_NC = 2
_NS = 16
_NW = _NC * _NS
_CHUNK = 80
_PER_W = E // _NW
_T = _PER_W // _CHUNK


_NBUF = 4


_XSTRIP = 624


def _gather_mul_body(x_hbm, ec_pos, ec_neg, out_pos, out_neg,
                     x_sh, idx, ra, rb, si, sga, sgb, swb):
    sid = lax.axis_index("s")
    wid = sid * _NC + lax.axis_index("c")
    w_base = wid * _PER_W

    @pl.when(sid < _NS - 1)
    def _():
        pltpu.sync_copy(x_hbm.at[pl.ds(sid * _XSTRIP, _XSTRIP)],
                        x_sh.at[pl.ds(sid * _XSTRIP, _XSTRIP)])


    @pl.when(sid == _NS - 1)
    def _():
        pltpu.sync_copy(x_hbm.at[pl.ds((_NS - 1) * _XSTRIP,
                                       N - (_NS - 1) * _XSTRIP)],
                        x_sh.at[pl.ds((_NS - 1) * _XSTRIP,
                                      N - (_NS - 1) * _XSTRIP)])

    plsc.subcore_barrier()

    def run_set(ec, out):
        def istart(t, b):
            base = w_base + t * _CHUNK
            pltpu.async_copy(ec.at[pl.ds(base, _CHUNK)], idx.at[b, 0],
                             si.at[b])
            pltpu.async_copy(ec.at[pl.ds(E + base, _CHUNK)], idx.at[b, 1],
                             si.at[b])

        def iwait(t, b):
            base = w_base + t * _CHUNK
            pltpu.make_async_copy(ec.at[pl.ds(base, _CHUNK)], idx.at[b, 0],
                                  si.at[b]).wait()
            pltpu.make_async_copy(ec.at[pl.ds(E + base, _CHUNK)],
                                  idx.at[b, 1], si.at[b]).wait()

        def gstart(b):
            pltpu.async_copy(x_sh.at[idx.at[b, 0]], ra.at[b], sga.at[b])
            pltpu.async_copy(x_sh.at[idx.at[b, 1]], rb.at[b], sgb.at[b])

        def gwait(b):
            pltpu.make_async_copy(x_sh.at[idx.at[b, 0]], ra.at[b],
                                  sga.at[b]).wait()
            pltpu.make_async_copy(x_sh.at[idx.at[b, 1]], rb.at[b],
                                  sgb.at[b]).wait()

        def wbwait(t, b):
            pltpu.make_async_copy(
                ra.at[b], out.at[pl.ds(w_base + t * _CHUNK, _CHUNK)],
                swb.at[b]).wait()

        def body(t, carry):
            b0 = lax.rem(t, _NBUF)
            b2 = lax.rem(t + 2, _NBUF)
            b3 = lax.rem(t + 3, _NBUF)

            @pl.when(t + 3 < _T)
            def _():
                istart(t + 3, b3)

            @pl.when(t + 2 < _T)
            def _():
                iwait(t + 2, b2)

                @pl.when(t >= 2)
                def _():
                    wbwait(t - 2, b2)

                gstart(b2)

            gwait(b0)

            def row_body(r, c):
                for k in range(D // 32):
                    sl = pl.ds(k * 16, 16)
                    w_a = ra[b0, r, sl]
                    w_b = rb[b0, r, sl]
                    topm = jnp.int32(-65536)
                    f32 = jnp.float32
                    alo = lax.bitcast_convert_type(lax.shift_left(w_a, 16), f32)
                    ahi = lax.bitcast_convert_type(w_a & topm, f32)
                    blo = lax.bitcast_convert_type(lax.shift_left(w_b, 16), f32)
                    bhi = lax.bitcast_convert_type(w_b & topm, f32)
                    plo = lax.bitcast_convert_type(alo * blo, jnp.int32)
                    phi = lax.bitcast_convert_type(ahi * bhi, jnp.int32)
                    half = jnp.int32(32768)
                    ra[b0, r, sl] = (
                        ((phi + half) & topm)
                        | lax.shift_right_logical(plo + half, 16))
                return c

            lax.fori_loop(0, _CHUNK, row_body, 0, unroll=4)

            pltpu.async_copy(ra.at[b0],
                             out.at[pl.ds(w_base + t * _CHUNK, _CHUNK)],
                             swb.at[b0])
            return carry

        for t in range(3):
            istart(t, t)
        for t in range(2):
            iwait(t, t)
            gstart(t)
        lax.fori_loop(0, _T, body, 0)
        for t in range(_T - _NBUF, _T):
            wbwait(t, t % _NBUF)

    run_set(ec_pos, out_pos)
    run_set(ec_neg, out_neg)


def _gather_mul(x, ei_pos, ei_neg):
    mesh = plsc.VectorSubcoreMesh(core_axis_name="c", subcore_axis_name="s")
    f = functools.partial(
        pl.kernel,
        mesh=mesh,
        out_type=[
            jax.ShapeDtypeStruct((E, D // 2), jnp.int32),
            jax.ShapeDtypeStruct((E, D // 2), jnp.int32),
        ],
        scratch_types=[
            pltpu.VMEM_SHARED((N, D // 2), jnp.int32),
            pltpu.VMEM((_NBUF, 2, _CHUNK), jnp.int32),
            pltpu.VMEM((_NBUF, _CHUNK, D // 2), jnp.int32),
            pltpu.VMEM((_NBUF, _CHUNK, D // 2), jnp.int32),
            pltpu.SemaphoreType.DMA((_NBUF,)),
            pltpu.SemaphoreType.DMA((_NBUF,)),
            pltpu.SemaphoreType.DMA((_NBUF,)),
            pltpu.SemaphoreType.DMA((_NBUF,)),
        ],
    )(_gather_mul_body)
    return f(x, ei_pos.reshape(2 * E), ei_neg.reshape(2 * E))


_B = 3200
_G = E // _B


def _decode_body(ep_ref, en_ref, wcat_ref, b1_ref, be1_ref, w2t_ref, b2_ref,
                 we2t_ref, be2_ref, attr_ref, pos_ref, neg_ref):
    wcat = wcat_ref[...]
    we2t = we2t_ref[...]
    be2 = be2_ref[...]
    h = jnp.maximum(ep_ref[...], jnp.bfloat16(0))
    a = lax.dot_general(h, wcat, (((1,), (0,)), ((), ())),
                        preferred_element_type=jnp.float32)
    a1 = jnp.maximum(a[:, :D] + b1_ref[...], 0.0)
    attr_ref[...] = jax.nn.sigmoid(jnp.dot(a1, w2t_ref[...]) + b2_ref[...])
    ae = jnp.maximum(a[:, D:] + be1_ref[...], 0.0)
    pos_ref[...] = jax.nn.sigmoid(jnp.dot(ae, we2t) + be2)
    hn = jnp.maximum(en_ref[...], jnp.bfloat16(0))
    an = jnp.maximum(
        lax.dot_general(hn, wcat[:, D:], (((1,), (0,)), ((), ())),
                        preferred_element_type=jnp.float32) + be1_ref[...],
        0.0)
    neg_ref[...] = jax.nn.sigmoid(jnp.dot(an, we2t) + be2)


def _decode(em_pos, em_neg, wcat_t, b1r, be1r, w2t8, b2r, we2t8, be2r):
    return pl.pallas_call(
        _decode_body,
        grid=(_G,),
        in_specs=[
            pl.BlockSpec((_B, D), lambda i: (i, 0)),
            pl.BlockSpec((_B, D), lambda i: (i, 0)),
            pl.BlockSpec((D, 2 * D), lambda i: (0, 0)),
            pl.BlockSpec((1, D), lambda i: (0, 0)),
            pl.BlockSpec((1, D), lambda i: (0, 0)),
            pl.BlockSpec((D, 8), lambda i: (0, 0)),
            pl.BlockSpec((1, 8), lambda i: (0, 0)),
            pl.BlockSpec((D, 8), lambda i: (0, 0)),
            pl.BlockSpec((1, 1), lambda i: (0, 0)),
        ],
        out_specs=[
            pl.BlockSpec((_B, 8), lambda i: (i, 0)),
            pl.BlockSpec((_B, 8), lambda i: (i, 0)),
            pl.BlockSpec((_B, 8), lambda i: (i, 0)),
        ],
        out_shape=[
            jax.ShapeDtypeStruct((E, 8), jnp.float32),
            jax.ShapeDtypeStruct((E, 8), jnp.float32),
            jax.ShapeDtypeStruct((E, 8), jnp.float32),
        ],
        compiler_params=pltpu.CompilerParams(
            dimension_semantics=("arbitrary",),
        ),
    )(em_pos, em_neg, wcat_t, b1r, be1r, w2t8, b2r, we2t8, be2r)


def kernel(x, edge_index, edge_index_neg, W1, b1, W2, b2, We1, be1, We2, be2):
    xp = lax.bitcast_convert_type(
        x.astype(jnp.bfloat16).reshape(N, D // 2, 2), jnp.int32)
    emp_i, emn_i = _gather_mul(xp, edge_index, edge_index_neg)
    em_pos = lax.bitcast_convert_type(emp_i, jnp.bfloat16).reshape(E, D)
    em_neg = lax.bitcast_convert_type(emn_i, jnp.bfloat16).reshape(E, D)

    wcat_t = jnp.concatenate([W1, We1], axis=0).T.astype(jnp.bfloat16)
    w2t8 = jnp.pad(W2, ((0, 1), (0, 0))).T
    b2r = jnp.pad(b2, (0, 1)).reshape(1, 8)
    we2t8 = jnp.pad(We2, ((0, 7), (0, 0))).T
    attr8, pos8, neg8 = _decode(
        em_pos, em_neg, wcat_t, b1.reshape(1, D), be1.reshape(1, D),
        w2t8, b2r, we2t8, be2.reshape(1, 1))
    return attr8[:, :7], pos8[:, 0], neg8[:, 0]
